# speedup vs baseline: 1.1835x; 1.1835x over previous
import functools

import jax
import jax.numpy as jnp
from jax.experimental import pallas as pl

D_MODEL = 1024
HID = 1024
N_ROUTED = 15
TOP_K = 3
EPS = 1e-09
RMS_EPS = 1.1920929e-07

N_TOK = 2048
RT = 256
FT = 1024
N_EXP = 16


def _router_body(x_ref, r_ref, xhat_ref, w_ref):
    x = x_ref[...]
    v = jnp.mean(x * x, axis=-1, keepdims=True)
    xhat_ref[...] = x * jax.lax.rsqrt(v + RMS_EPS)
    logits = jax.lax.dot_general(x, r_ref[...], (((1,), (0,)), ((), ())),
                                 preferred_element_type=jnp.float32)
    m = jnp.max(logits, axis=-1, keepdims=True)
    eg = jnp.exp(logits - m)
    gates = eg / jnp.sum(eg, axis=-1, keepdims=True)
    lanes = jax.lax.broadcasted_iota(jnp.int32, (RT, N_ROUTED), 1)
    g = gates
    sel = jnp.zeros((RT, N_ROUTED), dtype=jnp.bool_)
    for _ in range(TOP_K):
        j = jnp.argmax(g, axis=-1)[:, None]
        first = lanes == j
        sel = sel | first
        g = jnp.where(first, -1.0, g)
    masked = jnp.where(sel, gates, 0.0)
    w = masked / (jnp.sum(masked, axis=-1, keepdims=True) + EPS)
    w_ref[...] = jnp.concatenate(
        [w, jnp.ones((RT, 1), dtype=jnp.float32)], axis=-1)


def _ffn_body(xhat_ref, w_ref, W1_ref, W2_ref, out_ref):
    e = pl.program_id(1)
    xh = xhat_ref[...]
    h = jax.lax.dot_general(xh, W1_ref[0], (((1,), (1,)), ((), ())),
                            preferred_element_type=jnp.float32)
    h = h * jax.nn.sigmoid(h)
    y = jax.lax.dot_general(h, W2_ref[0], (((1,), (1,)), ((), ())),
                            preferred_element_type=jnp.float32)
    lanes = jax.lax.broadcasted_iota(jnp.int32, (1, N_EXP), 1)
    wcol = jnp.sum(jnp.where(lanes == e, w_ref[...], 0.0),
                   axis=-1, keepdims=True)
    contrib = y * wcol

    @pl.when(e == 0)
    def _():
        out_ref[...] = contrib

    @pl.when(e != 0)
    def _():
        out_ref[...] += contrib


@jax.jit
def kernel(x, router, W1_r, W2_r, g_r, W1_s, W2_s, g_s):
    B, T, _ = x.shape
    xf = x.reshape(B * T, D_MODEL)
    W1e = jnp.concatenate([W1_r * g_r[:, None, :], W1_s * g_s[:, None, :]], axis=0)
    W2e = jnp.concatenate([W2_r, W2_s], axis=0)

    xhat, w16 = pl.pallas_call(
        _router_body,
        grid=(N_TOK // RT,),
        in_specs=[
            pl.BlockSpec((RT, D_MODEL), lambda t: (t, 0)),
            pl.BlockSpec((D_MODEL, N_ROUTED), lambda t: (0, 0)),
        ],
        out_specs=[
            pl.BlockSpec((RT, D_MODEL), lambda t: (t, 0)),
            pl.BlockSpec((RT, N_EXP), lambda t: (t, 0)),
        ],
        out_shape=[
            jax.ShapeDtypeStruct((N_TOK, D_MODEL), jnp.float32),
            jax.ShapeDtypeStruct((N_TOK, N_EXP), jnp.float32),
        ],
    )(xf, router)

    out = pl.pallas_call(
        _ffn_body,
        grid=(N_TOK // FT, N_EXP),
        in_specs=[
            pl.BlockSpec((FT, D_MODEL), lambda t, e: (t, 0)),
            pl.BlockSpec((FT, N_EXP), lambda t, e: (t, 0)),
            pl.BlockSpec((1, HID, D_MODEL), lambda t, e: (e, 0, 0)),
            pl.BlockSpec((1, D_MODEL, HID), lambda t, e: (e, 0, 0)),
        ],
        out_specs=pl.BlockSpec((FT, D_MODEL), lambda t, e: (t, 0)),
        out_shape=jax.ShapeDtypeStruct((N_TOK, D_MODEL), jnp.float32),
    )(xhat, w16, W1e, W2e)

    return out.reshape(B, T, D_MODEL)
